# baseline (speedup 1.0000x reference)
import functools

import jax
import jax.numpy as jnp
from jax import lax
from jax.experimental import pallas as pl
from jax.experimental.pallas import tpu as pltpu
from jax.experimental.pallas import tpu_sc as plsc

N_USERS = 20000
N_ITEMS = 80000
N = N_USERS + N_ITEMS
E = 3200000
D = 6
DP = 8

NC, NS = 2, 16
NW = NC * NS
EPAD = 3276800
CHUNK = 3200
NBI = 4
NBR = 2
SEG_CHUNKS = (64, 0)
DEG_CHUNKS = (64, 0)

ACC_ROWS = 102400
RPT = ACC_ROWS // NS
DUMMY = N

R0, R1 = 50, 128
GRID = R0


def _mesh():
    return plsc.VectorSubcoreMesh(
        core_axis_name="c", subcore_axis_name="s",
        num_cores=NC, num_subcores=NS)


def _seg_pipeline(table, srcv, dstv, acc, sidx, didx, rows,
                  semi, semg, sems_, ebase, nchunk):
    def idx_start(k):
        b = k % NBI
        base = ebase + k * CHUNK
        return (
            pltpu.async_copy(srcv.at[pl.ds(base, CHUNK)], sidx.at[b], semi),
            pltpu.async_copy(dstv.at[pl.ds(base, CHUNK)], didx.at[b], semi),
        )

    pend_idx = {0: idx_start(0)}
    if nchunk > 1:
        pend_idx[1] = idx_start(1)
    pend_sc = {}
    for k in range(nchunk):
        bi = k % NBI
        br = k % NBR
        for cp in pend_idx.pop(k):
            cp.wait()
        if k - NBR in pend_sc:
            pend_sc.pop(k - NBR).wait()
        pltpu.async_copy(table.at[sidx.at[bi]], rows.at[br], semg).wait()
        pend_sc[k] = pltpu.async_copy(rows.at[br], acc.at[didx.at[bi]],
                                      sems_, add=True)
        if k + 2 < nchunk:
            pend_idx[k + 2] = idx_start(k + 2)
    for cp in pend_sc.values():
        cp.wait()


def _sc_segsum_body(table, srcv, dstv, zrows, out,
                    sidx, didx, rows, acc, semi, semg, sems_):
    c = lax.axis_index("c")
    s = lax.axis_index("s")
    pltpu.sync_copy(zrows, acc.at[pl.ds(s * RPT, RPT)])
    plsc.subcore_barrier()

    n0, n1 = SEG_CHUNKS
    core1_base = NS * n0 * CHUNK

    if n0:
        @pl.when(c == 0)
        def _():
            _seg_pipeline(table, srcv, dstv, acc, sidx, didx, rows,
                          semi, semg, sems_, s * (n0 * CHUNK), n0)

    if n1:
        @pl.when(c == 1)
        def _():
            _seg_pipeline(table, srcv, dstv, acc, sidx, didx, rows,
                          semi, semg, sems_, core1_base + s * (n1 * CHUNK), n1)

    plsc.subcore_barrier()
    pltpu.sync_copy(acc.at[pl.ds(s * RPT, RPT)], out.at[c, pl.ds(s * RPT, RPT)])


@functools.cache
def _sc_segsum():
    return pl.kernel(
        _sc_segsum_body,
        out_type=jax.ShapeDtypeStruct((NC, ACC_ROWS, DP), jnp.float32),
        mesh=_mesh(),
        compiler_params=pltpu.CompilerParams(use_tc_tiling_on_sc=False),
        scratch_types=[
            pltpu.VMEM((NBI, CHUNK), jnp.int32),
            pltpu.VMEM((NBI, CHUNK), jnp.int32),
            pltpu.VMEM((NBR, CHUNK, DP), jnp.float32),
            pltpu.VMEM_SHARED((ACC_ROWS, DP), jnp.float32),
            pltpu.SemaphoreType.DMA,
            pltpu.SemaphoreType.DMA,
            pltpu.SemaphoreType.DMA,
        ],
    )


def _deg_pipeline(dstv, acc, didx, ones_v, semi, sems_, ebase, nchunk):
    def idx_start(k):
        base = ebase + k * CHUNK
        return pltpu.async_copy(dstv.at[pl.ds(base, CHUNK)],
                                didx.at[k % NBI], semi)

    pend_idx = {0: idx_start(0)}
    if nchunk > 1:
        pend_idx[1] = idx_start(1)
    pend_sc = {}
    for k in range(nchunk):
        pend_idx.pop(k).wait()
        if k - 2 in pend_sc:
            pend_sc.pop(k - 2).wait()
        pend_sc[k] = pltpu.async_copy(ones_v, acc.at[didx.at[k % NBI]],
                                      sems_, add=True)
        if k + 2 < nchunk:
            pend_idx[k + 2] = idx_start(k + 2)
    for cp in pend_sc.values():
        cp.wait()


def _sc_degree_body(dstv, ones_hbm, zrows, out, didx, ones_v, acc, semi, sems_):
    c = lax.axis_index("c")
    s = lax.axis_index("s")
    pltpu.sync_copy(ones_hbm, ones_v)
    pltpu.sync_copy(zrows, acc.at[pl.ds(s * RPT, RPT)])
    plsc.subcore_barrier()

    n0, n1 = DEG_CHUNKS
    core1_base = NS * n0 * CHUNK

    if n0:
        @pl.when(c == 0)
        def _():
            _deg_pipeline(dstv, acc, didx, ones_v, semi, sems_,
                          s * (n0 * CHUNK), n0)

    if n1:
        @pl.when(c == 1)
        def _():
            _deg_pipeline(dstv, acc, didx, ones_v, semi, sems_,
                          core1_base + s * (n1 * CHUNK), n1)

    plsc.subcore_barrier()
    pltpu.sync_copy(acc.at[pl.ds(s * RPT, RPT)], out.at[c, pl.ds(s * RPT, RPT)])


@functools.cache
def _sc_degree():
    return pl.kernel(
        _sc_degree_body,
        out_type=jax.ShapeDtypeStruct((NC, ACC_ROWS, DP), jnp.float32),
        mesh=_mesh(),
        compiler_params=pltpu.CompilerParams(use_tc_tiling_on_sc=False),
        scratch_types=[
            pltpu.VMEM((NBI, CHUNK), jnp.int32),
            pltpu.VMEM((CHUNK, DP), jnp.float32),
            pltpu.VMEM_SHARED((ACC_ROWS, DP), jnp.float32),
            pltpu.SemaphoreType.DMA,
            pltpu.SemaphoreType.DMA,
        ],
    )


_bspec = pl.BlockSpec((1, R1, 128), lambda i: (i, 0, 0))
_pspec = pl.BlockSpec((NC, 1, R1, 128), lambda i: (0, i, 0, 0))
_wspec = pl.BlockSpec((128, 128), lambda i: (0, 0))
_bias_spec = pl.BlockSpec((1, 128), lambda i: (0, 0))
_flat = jax.ShapeDtypeStruct((R0, R1, 128), jnp.float32)


def _tc_prep_body(degp_ref, x0_ref, dinv_ref, g1_ref):
    deg = degp_ref[0, 0] + degp_ref[1, 0] + 1.0
    dinv = lax.rsqrt(deg)
    dinv_ref[0] = dinv
    g1_ref[0] = dinv * x0_ref[0]


def _tc_prep(degp, x0f):
    return pl.pallas_call(
        _tc_prep_body,
        grid=(GRID,),
        in_specs=[_pspec, _bspec],
        out_specs=[_bspec, _bspec],
        out_shape=[_flat] * 2,
    )(degp, x0f)


def _tc_layer_body(accp_ref, g_ref, dinv_ref, w_ref, b_ref, x_ref, gn_ref):
    s = accp_ref[0, 0] + accp_ref[1, 0] + g_ref[0]
    x = dinv_ref[0] * jnp.dot(s, w_ref[...],
                              preferred_element_type=jnp.float32) + b_ref[...]
    x_ref[0] = x
    gn_ref[0] = dinv_ref[0] * x


def _tc_layer(accp, g, dinv8, wblk, b128):
    return pl.pallas_call(
        _tc_layer_body,
        grid=(GRID,),
        in_specs=[_pspec, _bspec, _bspec, _wspec, _bias_spec],
        out_specs=[_bspec, _bspec],
        out_shape=[_flat] * 2,
    )(accp, g, dinv8, wblk, b128)


def _tc_final_body(accp_ref, g_ref, dinv_ref, w_ref, b_ref, x1_ref, x2_ref,
                   attw_ref, ml_ref, mr_ref, out_ref):
    s = accp_ref[0, 0] + accp_ref[1, 0] + g_ref[0]
    x3 = dinv_ref[0] * jnp.dot(s, w_ref[...],
                               preferred_element_type=jnp.float32) + b_ref[...]
    x1 = x1_ref[0]
    x2 = x2_ref[0]
    f0 = x1
    f1 = (x1 + x2) * 0.5
    f2 = (x1 + x2 + x3) * (1.0 / 3.0)
    attw = attw_ref[...]
    ml = ml_ref[...]
    mr = mr_ref[...]

    def mm(a, bmat):
        return jnp.dot(a, bmat, preferred_element_type=jnp.float32)

    hs = [mm(f0, attw), mm(f1, attw), mm(f2, attw)]
    hs.append(hs[2])
    us = [mm(h, ml) for h in hs[:3]]
    us.append(us[2])
    vs = [mm(h, mr) for h in hs[:3]]
    vs.append(vs[2])

    cols = [jnp.zeros_like(hs[0]) for _ in range(4)]
    for i in range(4):
        e = []
        for j in range(4):
            z = us[i] + vs[j]
            e.append(jnp.where(z > 0, z, 0.2 * z))
        m = jnp.maximum(jnp.maximum(e[0], e[1]), jnp.maximum(e[2], e[3]))
        p = [jnp.exp(ej - m) for ej in e]
        denom = p[0] + p[1] + p[2] + p[3]
        inv = 1.0 / denom
        for j in range(4):
            cols[j] = cols[j] + p[j] * inv

    out = cols[0] * hs[0]
    for j in range(1, 4):
        out = out + cols[j] * hs[j]
    out_ref[0] = out


def _tc_final(accp, g, dinv8, wblk, b128, x1, x2, attwblk, mlblk, mrblk):
    return pl.pallas_call(
        _tc_final_body,
        grid=(GRID,),
        in_specs=[_pspec, _bspec, _bspec, _wspec, _bias_spec,
                  _bspec, _bspec, _wspec, _wspec, _wspec],
        out_specs=_bspec,
        out_shape=_flat,
    )(accp, g, dinv8, wblk, b128, x1, x2, attwblk, mlblk, mrblk)


def kernel(user_preferences, item_ratings, edge_index, gcn_weights,
           gcn_biases, attW, attA):
    f32 = jnp.float32
    npad = EPAD - E
    srcv = jnp.concatenate([edge_index[0], jnp.zeros((npad,), jnp.int32)])
    dstv = jnp.concatenate([edge_index[1],
                            jnp.full((npad,), DUMMY, jnp.int32)])

    x0 = jnp.concatenate([user_preferences, item_ratings], axis=0)
    x0f = jnp.pad(x0, ((0, ACC_ROWS - N), (0, DP - D))).reshape(R0, R1, 128)

    eye16 = jnp.eye(16, dtype=f32)
    wpad = jnp.zeros((3, DP, DP), f32).at[:, :D, :D].set(gcn_weights)
    wblks = [jnp.kron(eye16, wpad[l]) for l in range(3)]
    b128s = [jnp.tile(jnp.pad(gcn_biases[l], (0, DP - D)), 16)[None, :]
             for l in range(3)]
    attwpad = jnp.zeros((DP, DP), f32).at[:D, :D].set(attW)
    attwblk = jnp.kron(eye16, attwpad)
    ones8 = jnp.ones((DP,), f32)
    aL = jnp.pad(attA[:D, 0], (0, DP - D))
    aR = jnp.pad(attA[D:, 0], (0, DP - D))
    mlblk = jnp.kron(eye16, jnp.outer(aL, ones8))
    mrblk = jnp.kron(eye16, jnp.outer(aR, ones8))

    zrows = jnp.zeros((RPT, DP), f32)
    ones_rows = jnp.ones((CHUNK, DP), f32)

    degp = _sc_degree()(dstv, ones_rows, zrows)
    dinv8, g = _tc_prep(degp.reshape(NC, R0, R1, 128), x0f)

    xs = []
    for l in range(3):
        accp = _sc_segsum()(g.reshape(ACC_ROWS, DP), srcv, dstv, zrows)
        accp = accp.reshape(NC, R0, R1, 128)
        if l < 2:
            x, g = _tc_layer(accp, g, dinv8, wblks[l], b128s[l])
            xs.append(x)
        else:
            final = _tc_final(accp, g, dinv8, wblks[l], b128s[l],
                              xs[0], xs[1], attwblk, mlblk, mrblk)

    final = final.reshape(ACC_ROWS, DP)[:N, :D]
    return final[:N_USERS], final[N_USERS:]


# speedup vs baseline: 1.2420x; 1.2420x over previous
import functools

import jax
import jax.numpy as jnp
from jax import lax
from jax.experimental import pallas as pl
from jax.experimental.pallas import tpu as pltpu
from jax.experimental.pallas import tpu_sc as plsc

N_USERS = 20000
N_ITEMS = 80000
N = N_USERS + N_ITEMS
E = 3200000
D = 6
DP = 8

NC, NS = 2, 16
NW = NC * NS
EPAD = 3276800
CHUNK = 3200
NBI = 4
NBR = 2
SEG_CHUNKS = (48, 16)
DEG_CHUNKS = (43, 21)

ACC_ROWS = 102400
RPT = ACC_ROWS // NS
DUMMY = N

R0, R1 = 50, 128
GRID = R0


def _mesh():
    return plsc.VectorSubcoreMesh(
        core_axis_name="c", subcore_axis_name="s",
        num_cores=NC, num_subcores=NS)


def _seg_pipeline(table, srcv, dstv, acc, sidx, didx, rows,
                  semi, semg, sems_, ebase, nchunk):
    def idx_start(k):
        b = k % NBI
        base = ebase + k * CHUNK
        return (
            pltpu.async_copy(srcv.at[pl.ds(base, CHUNK)], sidx.at[b], semi),
            pltpu.async_copy(dstv.at[pl.ds(base, CHUNK)], didx.at[b], semi),
        )

    pend_idx = {0: idx_start(0)}
    if nchunk > 1:
        pend_idx[1] = idx_start(1)
    pend_sc = {}
    for k in range(nchunk):
        bi = k % NBI
        br = k % NBR
        for cp in pend_idx.pop(k):
            cp.wait()
        if k - NBR in pend_sc:
            pend_sc.pop(k - NBR).wait()
        pltpu.async_copy(table.at[sidx.at[bi]], rows.at[br], semg).wait()
        pend_sc[k] = pltpu.async_copy(rows.at[br], acc.at[didx.at[bi]],
                                      sems_, add=True)
        if k + 2 < nchunk:
            pend_idx[k + 2] = idx_start(k + 2)
    for cp in pend_sc.values():
        cp.wait()


def _sc_segsum_body(table, srcv, dstv, zrows, out,
                    sidx, didx, rows, acc, semi, semg, sems_):
    c = lax.axis_index("c")
    s = lax.axis_index("s")
    pltpu.sync_copy(zrows, acc.at[pl.ds(s * RPT, RPT)])
    plsc.subcore_barrier()

    n0, n1 = SEG_CHUNKS
    core1_base = NS * n0 * CHUNK

    if n0:
        @pl.when(c == 0)
        def _():
            _seg_pipeline(table, srcv, dstv, acc, sidx, didx, rows,
                          semi, semg, sems_, s * (n0 * CHUNK), n0)

    if n1:
        @pl.when(c == 1)
        def _():
            _seg_pipeline(table, srcv, dstv, acc, sidx, didx, rows,
                          semi, semg, sems_, core1_base + s * (n1 * CHUNK), n1)

    plsc.subcore_barrier()
    pltpu.sync_copy(acc.at[pl.ds(s * RPT, RPT)], out.at[c, pl.ds(s * RPT, RPT)])


@functools.cache
def _sc_segsum():
    return pl.kernel(
        _sc_segsum_body,
        out_type=jax.ShapeDtypeStruct((NC, ACC_ROWS, DP), jnp.float32),
        mesh=_mesh(),
        compiler_params=pltpu.CompilerParams(use_tc_tiling_on_sc=False),
        scratch_types=[
            pltpu.VMEM((NBI, CHUNK), jnp.int32),
            pltpu.VMEM((NBI, CHUNK), jnp.int32),
            pltpu.VMEM((NBR, CHUNK, DP), jnp.float32),
            pltpu.VMEM_SHARED((ACC_ROWS, DP), jnp.float32),
            pltpu.SemaphoreType.DMA,
            pltpu.SemaphoreType.DMA,
            pltpu.SemaphoreType.DMA,
        ],
    )


def _deg_pipeline(dstv, acc, didx, ones_v, semi, sems_, ebase, nchunk):
    def idx_start(k):
        base = ebase + k * CHUNK
        return pltpu.async_copy(dstv.at[pl.ds(base, CHUNK)],
                                didx.at[k % NBI], semi)

    pend_idx = {0: idx_start(0)}
    if nchunk > 1:
        pend_idx[1] = idx_start(1)
    pend_sc = {}
    for k in range(nchunk):
        pend_idx.pop(k).wait()
        if k - 2 in pend_sc:
            pend_sc.pop(k - 2).wait()
        pend_sc[k] = pltpu.async_copy(ones_v, acc.at[didx.at[k % NBI]],
                                      sems_, add=True)
        if k + 2 < nchunk:
            pend_idx[k + 2] = idx_start(k + 2)
    for cp in pend_sc.values():
        cp.wait()


def _sc_degree_body(dstv, ones_hbm, zrows, out, didx, ones_v, acc, semi, sems_):
    c = lax.axis_index("c")
    s = lax.axis_index("s")
    pltpu.sync_copy(ones_hbm, ones_v)
    pltpu.sync_copy(zrows, acc.at[pl.ds(s * RPT, RPT)])
    plsc.subcore_barrier()

    n0, n1 = DEG_CHUNKS
    core1_base = NS * n0 * CHUNK

    if n0:
        @pl.when(c == 0)
        def _():
            _deg_pipeline(dstv, acc, didx, ones_v, semi, sems_,
                          s * (n0 * CHUNK), n0)

    if n1:
        @pl.when(c == 1)
        def _():
            _deg_pipeline(dstv, acc, didx, ones_v, semi, sems_,
                          core1_base + s * (n1 * CHUNK), n1)

    plsc.subcore_barrier()
    pltpu.sync_copy(acc.at[pl.ds(s * RPT, RPT)], out.at[c, pl.ds(s * RPT, RPT)])


@functools.cache
def _sc_degree():
    return pl.kernel(
        _sc_degree_body,
        out_type=jax.ShapeDtypeStruct((NC, ACC_ROWS, DP), jnp.float32),
        mesh=_mesh(),
        compiler_params=pltpu.CompilerParams(use_tc_tiling_on_sc=False),
        scratch_types=[
            pltpu.VMEM((NBI, CHUNK), jnp.int32),
            pltpu.VMEM((CHUNK, DP), jnp.float32),
            pltpu.VMEM_SHARED((ACC_ROWS, DP), jnp.float32),
            pltpu.SemaphoreType.DMA,
            pltpu.SemaphoreType.DMA,
        ],
    )


_bspec = pl.BlockSpec((1, R1, 128), lambda i: (i, 0, 0))
_pspec = pl.BlockSpec((NC, 1, R1, 128), lambda i: (0, i, 0, 0))
_wspec = pl.BlockSpec((128, 128), lambda i: (0, 0))
_bias_spec = pl.BlockSpec((1, 128), lambda i: (0, 0))
_flat = jax.ShapeDtypeStruct((R0, R1, 128), jnp.float32)


def _tc_prep_body(degp_ref, x0_ref, dinv_ref, g1_ref):
    deg = degp_ref[0, 0] + degp_ref[1, 0] + 1.0
    dinv = lax.rsqrt(deg)
    dinv_ref[0] = dinv
    g1_ref[0] = dinv * x0_ref[0]


def _tc_prep(degp, x0f):
    return pl.pallas_call(
        _tc_prep_body,
        grid=(GRID,),
        in_specs=[_pspec, _bspec],
        out_specs=[_bspec, _bspec],
        out_shape=[_flat] * 2,
    )(degp, x0f)


def _tc_layer_body(accp_ref, g_ref, dinv_ref, w_ref, b_ref, x_ref, gn_ref):
    s = accp_ref[0, 0] + accp_ref[1, 0] + g_ref[0]
    x = dinv_ref[0] * jnp.dot(s, w_ref[...],
                              preferred_element_type=jnp.float32) + b_ref[...]
    x_ref[0] = x
    gn_ref[0] = dinv_ref[0] * x


def _tc_layer(accp, g, dinv8, wblk, b128):
    return pl.pallas_call(
        _tc_layer_body,
        grid=(GRID,),
        in_specs=[_pspec, _bspec, _bspec, _wspec, _bias_spec],
        out_specs=[_bspec, _bspec],
        out_shape=[_flat] * 2,
    )(accp, g, dinv8, wblk, b128)


def _tc_final_body(accp_ref, g_ref, dinv_ref, w_ref, b_ref, x1_ref, x2_ref,
                   attw_ref, ml_ref, mr_ref, out_ref):
    s = accp_ref[0, 0] + accp_ref[1, 0] + g_ref[0]
    x3 = dinv_ref[0] * jnp.dot(s, w_ref[...],
                               preferred_element_type=jnp.float32) + b_ref[...]
    x1 = x1_ref[0]
    x2 = x2_ref[0]
    f0 = x1
    f1 = (x1 + x2) * 0.5
    f2 = (x1 + x2 + x3) * (1.0 / 3.0)
    attw = attw_ref[...]
    ml = ml_ref[...]
    mr = mr_ref[...]

    def mm(a, bmat):
        return jnp.dot(a, bmat, preferred_element_type=jnp.float32)

    hs = [mm(f0, attw), mm(f1, attw), mm(f2, attw)]
    hs.append(hs[2])
    us = [mm(h, ml) for h in hs[:3]]
    us.append(us[2])
    vs = [mm(h, mr) for h in hs[:3]]
    vs.append(vs[2])

    cols = [jnp.zeros_like(hs[0]) for _ in range(4)]
    for i in range(4):
        e = []
        for j in range(4):
            z = us[i] + vs[j]
            e.append(jnp.where(z > 0, z, 0.2 * z))
        m = jnp.maximum(jnp.maximum(e[0], e[1]), jnp.maximum(e[2], e[3]))
        p = [jnp.exp(ej - m) for ej in e]
        denom = p[0] + p[1] + p[2] + p[3]
        inv = 1.0 / denom
        for j in range(4):
            cols[j] = cols[j] + p[j] * inv

    out = cols[0] * hs[0]
    for j in range(1, 4):
        out = out + cols[j] * hs[j]
    out_ref[0] = out


def _tc_final(accp, g, dinv8, wblk, b128, x1, x2, attwblk, mlblk, mrblk):
    return pl.pallas_call(
        _tc_final_body,
        grid=(GRID,),
        in_specs=[_pspec, _bspec, _bspec, _wspec, _bias_spec,
                  _bspec, _bspec, _wspec, _wspec, _wspec],
        out_specs=_bspec,
        out_shape=_flat,
    )(accp, g, dinv8, wblk, b128, x1, x2, attwblk, mlblk, mrblk)


def kernel(user_preferences, item_ratings, edge_index, gcn_weights,
           gcn_biases, attW, attA):
    f32 = jnp.float32
    npad = EPAD - E
    srcv = jnp.concatenate([edge_index[0], jnp.zeros((npad,), jnp.int32)])
    dstv = jnp.concatenate([edge_index[1],
                            jnp.full((npad,), DUMMY, jnp.int32)])

    x0 = jnp.concatenate([user_preferences, item_ratings], axis=0)
    x0f = jnp.pad(x0, ((0, ACC_ROWS - N), (0, DP - D))).reshape(R0, R1, 128)

    eye16 = jnp.eye(16, dtype=f32)
    wpad = jnp.zeros((3, DP, DP), f32).at[:, :D, :D].set(gcn_weights)
    wblks = [jnp.kron(eye16, wpad[l]) for l in range(3)]
    b128s = [jnp.tile(jnp.pad(gcn_biases[l], (0, DP - D)), 16)[None, :]
             for l in range(3)]
    attwpad = jnp.zeros((DP, DP), f32).at[:D, :D].set(attW)
    attwblk = jnp.kron(eye16, attwpad)
    ones8 = jnp.ones((DP,), f32)
    aL = jnp.pad(attA[:D, 0], (0, DP - D))
    aR = jnp.pad(attA[D:, 0], (0, DP - D))
    mlblk = jnp.kron(eye16, jnp.outer(aL, ones8))
    mrblk = jnp.kron(eye16, jnp.outer(aR, ones8))

    zrows = jnp.zeros((RPT, DP), f32)
    ones_rows = jnp.ones((CHUNK, DP), f32)

    degp = _sc_degree()(dstv, ones_rows, zrows)
    dinv8, g = _tc_prep(degp.reshape(NC, R0, R1, 128), x0f)

    xs = []
    for l in range(3):
        accp = _sc_segsum()(g.reshape(ACC_ROWS, DP), srcv, dstv, zrows)
        accp = accp.reshape(NC, R0, R1, 128)
        if l < 2:
            x, g = _tc_layer(accp, g, dinv8, wblks[l], b128s[l])
            xs.append(x)
        else:
            final = _tc_final(accp, g, dinv8, wblks[l], b128s[l],
                              xs[0], xs[1], attwblk, mlblk, mrblk)

    final = final.reshape(ACC_ROWS, DP)[:N, :D]
    return final[:N_USERS], final[N_USERS:]
